# Initial kernel scaffold; baseline (speedup 1.0000x reference)
import jax
import jax.numpy as jnp
from jax.experimental import pallas as pl


def kernel(x, edge_index, edge_weight, W1, b1, W2, b2):
    raise NotImplementedError("write your pallas kernel here")


# speedup vs baseline: 13.3532x; 13.3532x over previous
import functools

import jax
import jax.numpy as jnp
from jax import lax
from jax.experimental import pallas as pl
from jax.experimental.pallas import tpu as pltpu
from jax.experimental.pallas import tpu_sc as plsc

N_NODES = 10000
D = 16
NC = 2
NS = 16
NW = NC * NS
CHUNK = 1024
ROWS_PER_CHUNK = CHUNK // 128


def _ceil_to(x, m):
    return (x + m - 1) // m * m


def _mm_body(x_ref, w_ref, o_ref):
    o_ref[...] = jnp.dot(x_ref[...], w_ref[...],
                         preferred_element_type=jnp.float32)


def _tc_matmul(x, w):
    n, _ = x.shape
    d = w.shape[1]
    return pl.pallas_call(
        _mm_body,
        out_shape=jax.ShapeDtypeStruct((n, d), jnp.float32),
    )(x, w)


def _combine_relu_mm_body(p_ref, b_ref, w_ref, o_ref):
    h = jax.nn.relu(p_ref[0] + p_ref[1] + b_ref[...])
    o_ref[...] = jnp.dot(h, w_ref[...], preferred_element_type=jnp.float32)


def _tc_combine_relu_matmul(parts, b_row, w):
    n = parts.shape[1]
    d = w.shape[1]
    return pl.pallas_call(
        _combine_relu_mm_body,
        out_shape=jax.ShapeDtypeStruct((n, d), jnp.float32),
    )(parts, b_row, w)


def _combine_body(p_ref, b_ref, o_ref):
    o_ref[...] = p_ref[0] + p_ref[1] + b_ref[...]


def _tc_combine(parts, b_row):
    n, d = parts.shape[1], parts.shape[2]
    return pl.pallas_call(
        _combine_body,
        out_shape=jax.ShapeDtypeStruct((n, d), jnp.float32),
    )(parts, b_row)


def _sc_edge_agg(hw, src2d, dst2d, w_pad, n_pad, chunks_per_worker):
    rows_per_tile = n_pad // NS

    mesh = plsc.VectorSubcoreMesh(core_axis_name="c", subcore_axis_name="s")

    @functools.partial(
        pl.kernel,
        out_type=jax.ShapeDtypeStruct((NC, n_pad, D), jnp.float32),
        mesh=mesh,
        compiler_params=pltpu.CompilerParams(use_tc_tiling_on_sc=False),
        scratch_types=[
            pltpu.VMEM((ROWS_PER_CHUNK, 128), jnp.int32),
            pltpu.VMEM((ROWS_PER_CHUNK, 128), jnp.int32),
            pltpu.VMEM((CHUNK,), jnp.float32),
            pltpu.VMEM((CHUNK, D), jnp.float32),
            pltpu.VMEM((rows_per_tile, D), jnp.float32),
            pltpu.VMEM_SHARED((n_pad, D), jnp.float32),
            pltpu.SemaphoreType.DMA,
        ],
    )
    def body(hw_hbm, src_hbm, dst_hbm, w_hbm, out_hbm,
             src_v, dst_v, w_v, rows_v, stage_v, acc_sh, sem):
        cid = lax.axis_index("c")
        sid = lax.axis_index("s")
        wid = cid * NS + sid

        zeros16 = jnp.zeros((16,), jnp.float32)

        def zrow(i, _):
            stage_v[i, :] = zeros16
            return 0

        lax.fori_loop(0, rows_per_tile, zrow, 0)
        pltpu.sync_copy(stage_v, acc_sh.at[pl.ds(sid * rows_per_tile,
                                                 rows_per_tile)])
        plsc.subcore_barrier()

        def chunk_body(c, _):
            row0 = (wid * chunks_per_worker + c) * ROWS_PER_CHUNK
            ebase = (wid * chunks_per_worker + c) * CHUNK
            pltpu.sync_copy(src_hbm.at[pl.ds(row0, ROWS_PER_CHUNK)], src_v)
            pltpu.sync_copy(dst_hbm.at[pl.ds(row0, ROWS_PER_CHUNK)], dst_v)
            pltpu.sync_copy(w_hbm.at[pl.ds(ebase, CHUNK)], w_v)

            copies = []
            for j in range(ROWS_PER_CHUNK):
                copies.append(pltpu.async_copy(
                    hw_hbm.at[src_v.at[j]],
                    rows_v.at[pl.ds(j * 128, 128)],
                    sem))
            for cp in copies:
                cp.wait()

            gd = lax.GatherDimensionNumbers(
                offset_dims=(), collapsed_slice_dims=(0,),
                start_index_map=(0,))

            def scale_group(g, _):
                base = g * 16
                wv = w_v[pl.ds(base, 16)]
                for r in range(16):
                    splat = lax.gather(
                        wv, jnp.full((16, 1), r, jnp.int32), gd, (1,),
                        mode=lax.GatherScatterMode.PROMISE_IN_BOUNDS)
                    rows_v[base + r, :] = rows_v[base + r, :] * splat
                return 0

            lax.fori_loop(0, CHUNK // 16, scale_group, 0)

            for j in range(ROWS_PER_CHUNK):
                pltpu.sync_copy(rows_v.at[pl.ds(j * 128, 128)],
                                acc_sh.at[dst_v.at[j]],
                                add=True)
            return 0

        lax.fori_loop(0, chunks_per_worker, chunk_body, 0)
        plsc.subcore_barrier()

        r0 = sid * rows_per_tile
        pltpu.sync_copy(acc_sh.at[pl.ds(r0, rows_per_tile)], stage_v)
        pltpu.sync_copy(stage_v, out_hbm.at[cid, pl.ds(r0, rows_per_tile)])

    return body(hw, src2d, dst2d, w_pad)


def kernel(x, edge_index, edge_weight, W1, b1, W2, b2):
    n, _ = x.shape
    e = edge_index.shape[1]
    d_out = W2.shape[1]
    n_pad = _ceil_to(n, 8 * NS)

    src = edge_index[0].astype(jnp.int32)
    dst = edge_index[1].astype(jnp.int32)
    w = edge_weight.astype(jnp.float32)

    e_pad = _ceil_to(e, NW * CHUNK)
    pad = e_pad - e
    if pad:
        src = jnp.concatenate([src, jnp.zeros((pad,), jnp.int32)])
        dst = jnp.concatenate([dst, jnp.zeros((pad,), jnp.int32)])
        w = jnp.concatenate([w, jnp.zeros((pad,), jnp.float32)])
    src2d = src.reshape(e_pad // 128, 128)
    dst2d = dst.reshape(e_pad // 128, 128)
    chunks_per_worker = e_pad // (NW * CHUNK)

    hw1 = _tc_matmul(x, W1)
    parts1 = _sc_edge_agg(hw1, src2d, dst2d, w, n_pad, chunks_per_worker)

    w2p = jnp.pad(W2, ((0, 0), (0, D - d_out)))
    b2p = jnp.pad(b2, (0, D - d_out))
    hw2 = _tc_combine_relu_matmul(parts1, b1.reshape(1, D), w2p)
    parts2 = _sc_edge_agg(hw2, src2d, dst2d, w, n_pad, chunks_per_worker)

    out16 = _tc_combine(parts2, b2p.reshape(1, D))
    return out16[:n, :d_out]
